# baseline (speedup 1.0000x reference)
import jax
import jax.numpy as jnp
from jax import lax
from jax.experimental import pallas as pl
from jax.experimental.pallas import tpu as pltpu
from jax.experimental.pallas import tpu_sc as plsc

_B, _T, _V, _L = 4, 256, 16, 8
_N = 40
_NC, _NS = 2, 16
_NW = _NC * _NS
_PAIRS_PER_W = (_B * _V) // _NW


def _lane(vec, idx):
    return jnp.where(idx == 0, vec[0],
                     jnp.where(idx == 1, vec[1],
                               jnp.where(idx == 2, vec[2], vec[3])))


def _sc_body(srl_hbm, vn_hbm, vlab_hbm, sll_hbm, sem_hbm, vlol_hbm, out_hbm,
             vlab_v, sll_v, sem_v, vlol_v, rows_v, res_v, sem):
    wid = lax.axis_index("s") * _NC + lax.axis_index("c")
    iota = lax.iota(jnp.int32, 16)

    stage = [
        pltpu.async_copy(vlab_hbm, vlab_v, sem),
        pltpu.async_copy(sll_hbm, sll_v, sem),
        pltpu.async_copy(sem_hbm, sem_v, sem),
        pltpu.async_copy(vlol_hbm, vlol_v, sem),
    ]
    for c in stage:
        c.wait()
    vlol = vlol_v[...]
    nrm = vlol[4] + vlol[5] + vlol[6] + vlol[7]
    inv_vec = 1.0 / jnp.full((16,), nrm.astype(jnp.float32))

    total = jnp.zeros((16,), jnp.float32)
    for j in range(_PAIRS_PER_W):
        pair = wid * _PAIRS_PER_W + j
        b = pair // _V
        v = pair - b * _V
        pairvec = jnp.full((16,), pair, jnp.int32)
        vlab = plsc.load_gather(vlab_v, [pairvec])[0]
        sll = plsc.load_gather(sll_v, [pairvec])[0]
        semrow = sem_v[pl.ds(pair * 16, 16)]

        copies = []
        for l in range(_L):
            copies.append(pltpu.async_copy(
                srl_hbm.at[b, vlab, semrow[l]],
                rows_v.at[pl.ds(l * _T, _T)], sem))
            copies.append(pltpu.async_copy(
                vn_hbm.at[b, vlab, semrow[8 + l]],
                rows_v.at[pl.ds((_L + l) * _T, _T)], sem))
        for c in copies:
            c.wait()

        v_ok = v < _lane(vlol, b)
        oln = jnp.where(b == 0, vlol[4],
                        jnp.where(b == 1, vlol[5],
                                  jnp.where(b == 2, vlol[6], vlol[7])))
        nchunks = (oln + 15) // 16

        for l in range(_L):
            r = semrow[l]
            a = semrow[8 + l]
            coef = ((l < sll) & v_ok).astype(jnp.float32)
            rmask = (r != 0).astype(jnp.float32)
            amask = (a != 0).astype(jnp.float32)

            def chunk_body(c, acc, l=l, rmask=rmask, amask=amask, oln=oln):
                t = c * 16 + iota
                x = rows_v[pl.ds(l * _T + c * 16, 16)] * rmask
                y = rows_v[pl.ds((_L + l) * _T + c * 16, 16)] * amask
                tm = (t < oln).astype(jnp.float32)
                return acc + jnp.abs(x - y) * tm

            acc_l = lax.fori_loop(0, nchunks, chunk_body,
                                  jnp.zeros((16,), jnp.float32))
            total = total + acc_l * coef

    res_v[...] = total * inv_vec
    pltpu.sync_copy(res_v, out_hbm.at[wid])


def kernel(log_srl, log_vn, v_label, v_l, orig_l, semlink, semlink_l):
    srl_t = jnp.transpose(log_srl, (0, 1, 3, 2))
    vn_t = jnp.transpose(log_vn, (0, 1, 3, 2))
    vlab1 = v_label.astype(jnp.int32).reshape(-1)
    sll1 = semlink_l.astype(jnp.int32).reshape(-1)
    sem1 = semlink.astype(jnp.int32).reshape(-1)
    vlol = jnp.concatenate([v_l.astype(jnp.int32), orig_l.astype(jnp.int32),
                            jnp.zeros((8,), jnp.int32)])

    sc_call = pl.kernel(
        _sc_body,
        out_type=jax.ShapeDtypeStruct((_NW, 16), jnp.float32),
        mesh=plsc.VectorSubcoreMesh(core_axis_name="c", subcore_axis_name="s"),
        scratch_types=[
            pltpu.VMEM((_B * _V,), jnp.int32),
            pltpu.VMEM((_B * _V,), jnp.int32),
            pltpu.VMEM((_B * _V * 2 * _L,), jnp.int32),
            pltpu.VMEM((16,), jnp.int32),
            pltpu.VMEM((2 * _L * _T,), jnp.float32),
            pltpu.VMEM((16,), jnp.float32),
            pltpu.SemaphoreType.DMA,
        ],
        compiler_params=pltpu.CompilerParams(
            needs_layout_passes=False,
            use_tc_tiling_on_sc=True,
        ),
    )
    partials = sc_call(srl_t, vn_t, vlab1, sll1, sem1, vlol)
    return jnp.sum(partials)


# speedup vs baseline: 1.0634x; 1.0634x over previous
import jax
import jax.numpy as jnp
from jax import lax
from jax.experimental import pallas as pl
from jax.experimental.pallas import tpu as pltpu
from jax.experimental.pallas import tpu_sc as plsc

_B, _T, _V, _L = 4, 256, 16, 8
_N = 40
_NC, _NS = 2, 16
_NW = _NC * _NS
_PAIRS_PER_W = (_B * _V) // _NW
_MROW = 32


def _sc_body(srl_hbm, vn_hbm, meta_hbm, out_hbm,
             meta0_v, meta1_v, rows0_v, rows1_v, res_v, sem0, sem1):
    wid = lax.axis_index("s") * _NC + lax.axis_index("c")
    iota = lax.iota(jnp.int32, 16)

    pair0 = wid * _PAIRS_PER_W
    m_copies = [pltpu.async_copy(meta_hbm.at[pair0], meta0_v, sem0),
                pltpu.async_copy(meta_hbm.at[pair0 + 1], meta1_v, sem1)]
    for c in m_copies:
        c.wait()

    total = jnp.zeros((16,), jnp.float32)
    inv_vec = jnp.zeros((16,), jnp.float32)
    for j, (meta_v, rows_v, sem) in enumerate(
            [(meta0_v, rows0_v, sem0), (meta1_v, rows1_v, sem1)]):
        pair = pair0 + j
        b = pair // _V
        v = pair - b * _V
        m0 = meta_v[pl.ds(0, 16)]
        m1 = meta_v[pl.ds(16, 16)]

        vlab = m0[0]
        sll = m0[1]
        v_ok = v < m0[2]
        live = v_ok & (sll > 0)
        oln = jnp.where(b == 0, m0[3],
                        jnp.where(b == 1, m0[4],
                                  jnp.where(b == 2, m0[5], m0[6])))
        nrm = m0[3] + m0[4] + m0[5] + m0[6]
        inv_vec = 1.0 / jnp.full((16,), nrm.astype(jnp.float32))
        nchunks = (oln + 15) // 16

        @pl.when(live)
        def _fire(srl_hbm=srl_hbm, vn_hbm=vn_hbm, rows_v=rows_v, sem=sem,
                  b=b, vlab=vlab, m0=m0, m1=m1):
            copies = []
            for l in range(_L):
                copies.append(pltpu.async_copy(
                    srl_hbm.at[b, vlab, m0[8 + l]],
                    rows_v.at[pl.ds(l * _T, _T)], sem))
                copies.append(pltpu.async_copy(
                    vn_hbm.at[b, vlab, m1[l]],
                    rows_v.at[pl.ds((_L + l) * _T, _T)], sem))
            for c in copies:
                c.wait()

        for l in range(_L):
            r = m0[8 + l]
            a = m1[l]
            use = (l < sll) & v_ok & ((r != 0) | (a != 0))
            rmask = (r != 0).astype(jnp.float32)
            amask = (a != 0).astype(jnp.float32)
            trip = jnp.where(use, nchunks, 0)

            def chunk_body(c, acc, l=l, rows_v=rows_v,
                           rmask=rmask, amask=amask, oln=oln):
                t = c * 16 + iota
                x = rows_v[pl.ds(l * _T + c * 16, 16)] * rmask
                y = rows_v[pl.ds((_L + l) * _T + c * 16, 16)] * amask
                tm = (t < oln).astype(jnp.float32)
                return acc + jnp.abs(x - y) * tm

            total = lax.fori_loop(0, trip, chunk_body, total)

    res_v[...] = total * inv_vec
    pltpu.sync_copy(res_v, out_hbm.at[wid])


def kernel(log_srl, log_vn, v_label, v_l, orig_l, semlink, semlink_l):
    srl_t = jnp.transpose(log_srl, (0, 1, 3, 2))
    vn_t = jnp.transpose(log_vn, (0, 1, 3, 2))

    bv = _B * _V
    meta = jnp.concatenate([
        v_label.astype(jnp.int32).reshape(bv, 1),
        semlink_l.astype(jnp.int32).reshape(bv, 1),
        jnp.broadcast_to(v_l.astype(jnp.int32)[:, None, None],
                         (_B, _V, 1)).reshape(bv, 1),
        jnp.broadcast_to(orig_l.astype(jnp.int32)[None, :], (bv, _B)),
        jnp.zeros((bv, 1), jnp.int32),
        semlink.astype(jnp.int32).reshape(bv, 2 * _L),
        jnp.zeros((bv, _MROW - 24), jnp.int32),
    ], axis=1)

    sc_call = pl.kernel(
        _sc_body,
        out_type=jax.ShapeDtypeStruct((_NW, 16), jnp.float32),
        mesh=plsc.VectorSubcoreMesh(core_axis_name="c", subcore_axis_name="s"),
        scratch_types=[
            pltpu.VMEM((_MROW,), jnp.int32),
            pltpu.VMEM((_MROW,), jnp.int32),
            pltpu.VMEM((2 * _L * _T,), jnp.float32),
            pltpu.VMEM((2 * _L * _T,), jnp.float32),
            pltpu.VMEM((16,), jnp.float32),
            pltpu.SemaphoreType.DMA,
            pltpu.SemaphoreType.DMA,
        ],
        compiler_params=pltpu.CompilerParams(
            needs_layout_passes=False,
            use_tc_tiling_on_sc=True,
        ),
    )
    partials = sc_call(srl_t, vn_t, meta)
    return jnp.sum(partials)


# speedup vs baseline: 1.0987x; 1.0332x over previous
import jax
import jax.numpy as jnp
from jax import lax
from jax.experimental import pallas as pl
from jax.experimental.pallas import tpu as pltpu
from jax.experimental.pallas import tpu_sc as plsc

_B, _T, _V, _L = 4, 256, 16, 8
_N = 40
_NC, _NS = 2, 16
_NW = _NC * _NS
_PAIRS_PER_W = (_B * _V) // _NW


def _sc_body(srl_hbm, vn_hbm, vlab_hbm, sll_hbm, sem_hbm, vl_hbm, ol_hbm,
             out_hbm, vlab_v, sll_v, sem_v, vl_v, ol_v,
             rows0_v, rows1_v, res_v, sem0, sem1):
    wid = lax.axis_index("s") * _NC + lax.axis_index("c")
    iota = lax.iota(jnp.int32, 16)
    pair0 = wid * _PAIRS_PER_W
    b = pair0 // _V
    v0 = pair0 - b * _V

    stage = [
        pltpu.async_copy(vlab_hbm.at[b], vlab_v, sem0),
        pltpu.async_copy(sll_hbm.at[b], sll_v, sem0),
        pltpu.async_copy(sem_hbm.at[b], sem_v, sem0),
        pltpu.async_copy(vl_hbm, vl_v.at[pl.ds(0, _B)], sem0),
        pltpu.async_copy(ol_hbm, ol_v.at[pl.ds(0, _B)], sem0),
    ]
    for c in stage:
        c.wait()

    bvec = jnp.full((16,), b, jnp.int32)
    oln = plsc.load_gather(ol_v, [bvec])[0]
    vl_b = plsc.load_gather(vl_v, [bvec])[0]
    olv = ol_v[...]
    nrm = olv[0] + olv[1] + olv[2] + olv[3]
    inv_vec = 1.0 / jnp.full((16,), nrm.astype(jnp.float32))
    nchunks = (oln + 15) // 16
    kvec = iota // 8
    lvec = iota - kvec * 8

    semrows = []
    vlabs = []
    slls = []
    fired = []
    for j, (rows_v, sem) in enumerate([(rows0_v, sem0), (rows1_v, sem1)]):
        v = v0 + j
        vvec = jnp.full((16,), v, jnp.int32)
        vlab = plsc.load_gather(vlab_v, [vvec])[0]
        semrow = plsc.load_gather(sem_v, [kvec, lvec, vvec])
        vlabs.append(vlab)
        slls.append(plsc.load_gather(sll_v, [vvec])[0])
        semrows.append(semrow)
        copies = []
        for l in range(_L):
            copies.append(pltpu.async_copy(
                srl_hbm.at[b, vlab, semrow[l]],
                rows_v.at[pl.ds(l * _T, _T)], sem))
            copies.append(pltpu.async_copy(
                vn_hbm.at[b, vlab, semrow[8 + l]],
                rows_v.at[pl.ds((_L + l) * _T, _T)], sem))
        fired.append(copies)

    total = jnp.zeros((16,), jnp.float32)
    for j, (rows_v, sem) in enumerate([(rows0_v, sem0), (rows1_v, sem1)]):
        v = v0 + j
        semrow = semrows[j]
        sll = slls[j]
        v_ok = v < vl_b
        for c in fired[j]:
            c.wait()

        for l in range(_L):
            r = semrow[l]
            a = semrow[8 + l]
            use = (l < sll) & v_ok & ((r != 0) | (a != 0))
            rmask = (r != 0).astype(jnp.float32)
            amask = (a != 0).astype(jnp.float32)
            trip = jnp.where(use, nchunks, 0)

            def chunk_body(c, acc, l=l, rows_v=rows_v,
                           rmask=rmask, amask=amask, oln=oln):
                t = c * 16 + iota
                x = rows_v[pl.ds(l * _T + c * 16, 16)] * rmask
                y = rows_v[pl.ds((_L + l) * _T + c * 16, 16)] * amask
                tm = (t < oln).astype(jnp.float32)
                return acc + jnp.abs(x - y) * tm

            total = lax.fori_loop(0, trip, chunk_body, total)

    res_v[...] = total * inv_vec
    pltpu.sync_copy(res_v, out_hbm.at[wid])


def kernel(log_srl, log_vn, v_label, v_l, orig_l, semlink, semlink_l):
    srl_t = jnp.transpose(log_srl, (0, 1, 3, 2))
    vn_t = jnp.transpose(log_vn, (0, 1, 3, 2))
    sem_t = jnp.transpose(semlink.astype(jnp.int32), (0, 2, 3, 1))

    sc_call = pl.kernel(
        _sc_body,
        out_type=jax.ShapeDtypeStruct((_NW, 16), jnp.float32),
        mesh=plsc.VectorSubcoreMesh(core_axis_name="c", subcore_axis_name="s"),
        scratch_types=[
            pltpu.VMEM((_V,), jnp.int32),
            pltpu.VMEM((_V,), jnp.int32),
            pltpu.VMEM((2, _L, _V), jnp.int32),
            pltpu.VMEM((16,), jnp.int32),
            pltpu.VMEM((16,), jnp.int32),
            pltpu.VMEM((2 * _L * _T,), jnp.float32),
            pltpu.VMEM((2 * _L * _T,), jnp.float32),
            pltpu.VMEM((16,), jnp.float32),
            pltpu.SemaphoreType.DMA,
            pltpu.SemaphoreType.DMA,
        ],
        compiler_params=pltpu.CompilerParams(
            needs_layout_passes=False,
            use_tc_tiling_on_sc=True,
        ),
    )
    partials = sc_call(srl_t, vn_t,
                       v_label.astype(jnp.int32), semlink_l.astype(jnp.int32),
                       sem_t, v_l.astype(jnp.int32), orig_l.astype(jnp.int32))
    return jnp.sum(partials)


# speedup vs baseline: 1.1925x; 1.0854x over previous
import jax
import jax.numpy as jnp
from jax import lax
from jax.experimental import pallas as pl
from jax.experimental.pallas import tpu as pltpu
from jax.experimental.pallas import tpu_sc as plsc

_B, _T, _V, _L = 4, 256, 16, 8
_N = 40
_NC, _NS = 2, 16
_NW = _NC * _NS
_PAIRS_PER_W = (_B * _V) // _NW


_DNUMS = lax.GatherDimensionNumbers(
    offset_dims=(), collapsed_slice_dims=(0,), start_index_map=(0,))


def _take(vec, idx):
    return lax.gather(vec, idx[:, None], _DNUMS, (1,),
                      mode=lax.GatherScatterMode.PROMISE_IN_BOUNDS)


def _sc_body(srl_hbm, vn_hbm, vlab_hbm, sll_hbm, sem_hbm, vl_hbm, ol_hbm,
             out_hbm, vlab_v, sll_v, sem_v, vl_v, ol_v, rows_v, res_v, sem):
    wid = lax.axis_index("s") * _NC + lax.axis_index("c")
    iota = lax.iota(jnp.int32, 16)
    pair0 = wid * _PAIRS_PER_W
    b = pair0 // _V
    v0 = pair0 - b * _V

    stage = [
        pltpu.async_copy(vlab_hbm.at[b], vlab_v, sem),
        pltpu.async_copy(sll_hbm.at[b], sll_v, sem),
        pltpu.async_copy(sem_hbm.at[b], sem_v, sem),
        pltpu.async_copy(vl_hbm, vl_v.at[pl.ds(0, _B)], sem),
        pltpu.async_copy(ol_hbm, ol_v.at[pl.ds(0, _B)], sem),
    ]
    for c in stage:
        c.wait()

    bvec = jnp.full((16,), b, jnp.int32)
    oln = plsc.load_gather(ol_v, [bvec])[0]
    vl_b = plsc.load_gather(vl_v, [bvec])[0]
    olv = ol_v[...]
    nrm = olv[0] + olv[1] + olv[2] + olv[3]
    inv_vec = 1.0 / jnp.full((16,), nrm.astype(jnp.float32))
    nchunks = (oln + 15) // 16

    j_u = iota // 8
    l_u = iota - j_u * 8
    v0vec = jnp.full((16,), v0, jnp.int32)
    sr0 = plsc.load_gather(sem_v, [j_u, l_u, v0vec])
    sr1 = plsc.load_gather(sem_v, [j_u, l_u, v0vec + 1])
    vlab0 = plsc.load_gather(vlab_v, [v0vec])[0]
    vlab1 = plsc.load_gather(vlab_v, [v0vec + 1])[0]
    sll0 = plsc.load_gather(sll_v, [v0vec])[0]
    sll1 = plsc.load_gather(sll_v, [v0vec + 1])[0]

    rvec = jnp.where(iota < 8, _take(sr0, l_u), _take(sr1, l_u))
    avec = jnp.where(iota < 8, _take(sr0, l_u + 8), _take(sr1, l_u + 8))
    sll_u = jnp.where(iota < 8, jnp.full((16,), sll0, jnp.int32),
                      jnp.full((16,), sll1, jnp.int32))
    use_u = (l_u < sll_u) & (v0 + j_u < vl_b) & ((rvec != 0) | (avec != 0))
    trip_u = jnp.where(use_u, jnp.full((16,), nchunks, jnp.int32), 0)
    rm_u = (rvec != 0).astype(jnp.float32)
    am_u = (avec != 0).astype(jnp.float32)
    xoff_u = (iota + j_u * 8) * _T

    def fire_srl(u, _):
        uvec = jnp.full((16,), u, jnp.int32)
        r = _take(rvec, uvec)[0]
        off = pl.multiple_of(_take(xoff_u, uvec)[0], _T)
        vlab = jnp.where(u < 8, vlab0, vlab1)
        pltpu.async_copy(srl_hbm.at[b, vlab, r],
                         rows_v.at[pl.ds(off, _T)], sem)
        return 0

    def fire_vn(u, _):
        uvec = jnp.full((16,), u, jnp.int32)
        a = _take(avec, uvec)[0]
        off = pl.multiple_of(_take(xoff_u, uvec)[0], _T) + _L * _T
        vlab = jnp.where(u < 8, vlab0, vlab1)
        pltpu.async_copy(vn_hbm.at[b, vlab, a],
                         rows_v.at[pl.ds(off, _T)], sem)
        return 0

    lax.fori_loop(0, 16, fire_srl, 0)
    lax.fori_loop(0, 16, fire_vn, 0)

    def drain(i, _):
        pltpu.make_async_copy(srl_hbm.at[0, 0, 0],
                              rows_v.at[pl.ds(0, _T)], sem).wait()
        return 0

    lax.fori_loop(0, 32, drain, 0)

    def u_body(u, tot):
        uvec = jnp.full((16,), u, jnp.int32)
        trip = _take(trip_u, uvec)[0]
        rm = _take(rm_u, uvec)[0]
        am = _take(am_u, uvec)[0]
        xo = pl.multiple_of(_take(xoff_u, uvec)[0], _T)
        yo = xo + _L * _T

        def chunk(c, acc):
            t = c * 16 + iota
            x = rows_v[pl.ds(xo + c * 16, 16)] * rm
            y = rows_v[pl.ds(yo + c * 16, 16)] * am
            tm = (t < oln).astype(jnp.float32)
            return acc + jnp.abs(x - y) * tm

        return lax.fori_loop(0, trip, chunk, tot)

    total = lax.fori_loop(0, 16, u_body, jnp.zeros((16,), jnp.float32))

    res_v[...] = total * inv_vec
    pltpu.sync_copy(res_v, out_hbm.at[wid])


def kernel(log_srl, log_vn, v_label, v_l, orig_l, semlink, semlink_l):
    srl_t = jnp.transpose(log_srl, (0, 1, 3, 2))
    vn_t = jnp.transpose(log_vn, (0, 1, 3, 2))
    sem_t = jnp.transpose(semlink.astype(jnp.int32), (0, 2, 3, 1))

    sc_call = pl.kernel(
        _sc_body,
        out_type=jax.ShapeDtypeStruct((_NW, 16), jnp.float32),
        mesh=plsc.VectorSubcoreMesh(core_axis_name="c", subcore_axis_name="s"),
        scratch_types=[
            pltpu.VMEM((_V,), jnp.int32),
            pltpu.VMEM((_V,), jnp.int32),
            pltpu.VMEM((2, _L, _V), jnp.int32),
            pltpu.VMEM((16,), jnp.int32),
            pltpu.VMEM((16,), jnp.int32),
            pltpu.VMEM((2 * 2 * _L * _T,), jnp.float32),
            pltpu.VMEM((16,), jnp.float32),
            pltpu.SemaphoreType.DMA,
        ],
        compiler_params=pltpu.CompilerParams(
            needs_layout_passes=False,
            use_tc_tiling_on_sc=True,
        ),
    )
    partials = sc_call(srl_t, vn_t,
                       v_label.astype(jnp.int32), semlink_l.astype(jnp.int32),
                       sem_t, v_l.astype(jnp.int32), orig_l.astype(jnp.int32))
    return jnp.sum(partials)
